# baseline (speedup 1.0000x reference)
import functools

import jax
import jax.numpy as jnp
from jax import lax
from jax.experimental import pallas as pl
from jax.experimental.pallas import tpu as pltpu
from jax.experimental.pallas import tpu_sc as plsc

_HIDDEN = 128
_B = 4096
_L = 50
_N_ROWS = _B * _L

_INFO = plsc.get_sparse_core_info()
_NC = _INFO.num_cores
_NS = _INFO.num_subcores
_NW = _NC * _NS
_PER_W = _N_ROWS // _NW
_SEQ_PER_W = _PER_W // _L
_CHUNK = 640
_N_CHUNKS = _PER_W // _CHUNK
_SLABS = _CHUNK // _HIDDEN


def _sc_gather_body(idx_hbm, table_hbm, out_hbm, idx2_v, idx_v, rows_v, sem):
    wid = lax.axis_index("s") * _NC + lax.axis_index("c")

    pltpu.sync_copy(idx_hbm.at[pl.ds(wid * _SEQ_PER_W, _SEQ_PER_W), :], idx2_v)

    def flatten(j, carry):
        k = j * 16 + lax.iota(jnp.int32, 16)
        r = lax.shift_right_logical(k * 41944, 21)
        col = k - r * _L
        idx_v[pl.ds(j * 16, 16)] = plsc.load_gather(idx2_v, [r, col])
        return carry

    lax.fori_loop(0, _PER_W // 16, flatten, 0, unroll=8)

    def chunk(c, carry):
        off = c * _CHUNK
        pltpu.async_copy(
            table_hbm.at[idx_v.at[pl.ds(off, _CHUNK)]], rows_v, sem
        ).wait()
        pltpu.sync_copy(
            rows_v.reshape(_SLABS, _HIDDEN, _HIDDEN),
            out_hbm.at[pl.ds(wid * (_PER_W // _HIDDEN) + c * _SLABS, _SLABS)],
        )
        return carry

    lax.fori_loop(0, _N_CHUNKS, chunk, 0)


_sc_gather = functools.partial(
    pl.kernel,
    mesh=plsc.VectorSubcoreMesh(core_axis_name="c", subcore_axis_name="s"),
    out_type=jax.ShapeDtypeStruct((_N_ROWS // _HIDDEN, _HIDDEN, _HIDDEN),
                                  jnp.float32),
    scratch_types=[
        pltpu.VMEM((_SEQ_PER_W, _L), jnp.int32),
        pltpu.VMEM((_PER_W,), jnp.int32),
        pltpu.VMEM((_CHUNK, _HIDDEN), jnp.float32),
        pltpu.SemaphoreType.DMA,
    ],
    compiler_params=pltpu.CompilerParams(
        needs_layout_passes=False, use_tc_tiling_on_sc=True
    ),
)(_sc_gather_body)


_BB = 64
_BLK = _BB * _L


def _mlp_body(x_ref, w1_ref, b1_ref, w2_ref, b2_ref, o_ref):
    x = x_ref[...].reshape(_BLK, _HIDDEN)
    h = jnp.dot(x, w1_ref[...], preferred_element_type=jnp.float32)
    h = jnp.maximum(h + b1_ref[...], 0.0)
    o = jnp.dot(h, w2_ref[...], preferred_element_type=jnp.float32)
    o_ref[...] = jnp.maximum(o + b2_ref[...], 0.0).reshape(o_ref.shape)


def _mlp(x, W1, b1, W2, b2):
    code = W2.shape[1]
    return pl.pallas_call(
        _mlp_body,
        grid=(_B // _BB,),
        in_specs=[
            pl.BlockSpec((_BLK // _HIDDEN, _HIDDEN, _HIDDEN),
                         lambda i: (i, 0, 0)),
            pl.BlockSpec((_HIDDEN, _HIDDEN), lambda i: (0, 0)),
            pl.BlockSpec((1, _HIDDEN), lambda i: (0, 0)),
            pl.BlockSpec((_HIDDEN, code), lambda i: (0, 0)),
            pl.BlockSpec((1, code), lambda i: (0, 0)),
        ],
        out_specs=pl.BlockSpec((_BB, _L, code), lambda i: (i, 0, 0)),
        out_shape=jax.ShapeDtypeStruct((_B, _L, code), jnp.float32),
    )(x, W1, b1[None, :], W2, b2[None, :])


def kernel(src_seq, emb_table, W1, b1, W2, b2):
    B, L = src_seq.shape
    gathered = _sc_gather(src_seq, emb_table)
    return _mlp(gathered, W1, b1, W2, b2)


# speedup vs baseline: 1.0542x; 1.0542x over previous
import functools

import jax
import jax.numpy as jnp
from jax import lax
from jax.experimental import pallas as pl
from jax.experimental.pallas import tpu as pltpu
from jax.experimental.pallas import tpu_sc as plsc

_HIDDEN = 128
_B = 4096
_L = 50
_N_ROWS = _B * _L

_G = 2
_BPG = _B // _G
_ROWS_G = _BPG * _L

_INFO = plsc.get_sparse_core_info()
_NC = _INFO.num_cores
_NS = _INFO.num_subcores
_NW = _NC * _NS
_PER_W = _ROWS_G // _NW
_SEQ_PER_W = _PER_W // _L
_CHUNK = 640
_N_CHUNKS = _PER_W // _CHUNK
_SLABS = _CHUNK // _HIDDEN


def _make_sc_gather(go):
    def body(idx_hbm, table_hbm, out_hbm, idx2_v, idx_v, rows_v, sem):
        wid = lax.axis_index("s") * _NC + lax.axis_index("c")

        seq_base = go * _BPG + wid * _SEQ_PER_W
        pltpu.sync_copy(idx_hbm.at[pl.ds(seq_base, _SEQ_PER_W), :], idx2_v)

        def flatten(j, carry):
            k = j * 16 + lax.iota(jnp.int32, 16)
            r = lax.shift_right_logical(k * 41944, 21)
            col = k - r * _L
            idx_v[pl.ds(j * 16, 16)] = plsc.load_gather(idx2_v, [r, col])
            return carry

        lax.fori_loop(0, _PER_W // 16, flatten, 0, unroll=8)

        def chunk(c, carry):
            off = c * _CHUNK
            pltpu.async_copy(
                table_hbm.at[idx_v.at[pl.ds(off, _CHUNK)]], rows_v, sem
            ).wait()
            pltpu.sync_copy(
                rows_v.reshape(_SLABS, _HIDDEN, _HIDDEN),
                out_hbm.at[
                    pl.ds(wid * (_PER_W // _HIDDEN) + c * _SLABS, _SLABS)
                ],
            )
            return carry

        lax.fori_loop(0, _N_CHUNKS, chunk, 0)

    return functools.partial(
        pl.kernel,
        mesh=plsc.VectorSubcoreMesh(core_axis_name="c", subcore_axis_name="s"),
        out_type=jax.ShapeDtypeStruct((_ROWS_G // _HIDDEN, _HIDDEN, _HIDDEN),
                                      jnp.float32),
        scratch_types=[
            pltpu.VMEM((_SEQ_PER_W, _L), jnp.int32),
            pltpu.VMEM((_PER_W,), jnp.int32),
            pltpu.VMEM((_CHUNK, _HIDDEN), jnp.float32),
            pltpu.SemaphoreType.DMA,
        ],
        compiler_params=pltpu.CompilerParams(
            needs_layout_passes=False, use_tc_tiling_on_sc=True
        ),
    )(body)


_SC_GATHERS = [_make_sc_gather(go) for go in range(_G)]

_BB = 64
_BLK = _BB * _L


def _mlp_compute(x_ref, w1_ref, b1_ref, w2_ref, b2_ref, o_ref):
    x = x_ref[...].reshape(_BLK, _HIDDEN)
    h = jnp.dot(x, w1_ref[...], preferred_element_type=jnp.float32)
    h = jnp.maximum(h + b1_ref[...], 0.0)
    o = jnp.dot(h, w2_ref[...], preferred_element_type=jnp.float32)
    o_ref[...] = jnp.maximum(o + b2_ref[...], 0.0).reshape(o_ref.shape)


def _mlp_body_first(x_ref, w1_ref, b1_ref, w2_ref, b2_ref, o_ref):
    _mlp_compute(x_ref, w1_ref, b1_ref, w2_ref, b2_ref, o_ref)


def _mlp_body_acc(x_ref, w1_ref, b1_ref, w2_ref, b2_ref, acc_ref, o_ref):
    del acc_ref
    _mlp_compute(x_ref, w1_ref, b1_ref, w2_ref, b2_ref, o_ref)


def _mlp(go, x, W1, b1, W2, b2, acc=None):
    code = W2.shape[1]
    in_specs = [
        pl.BlockSpec((_BLK // _HIDDEN, _HIDDEN, _HIDDEN),
                     lambda i: (i, 0, 0)),
        pl.BlockSpec((_HIDDEN, _HIDDEN), lambda i: (0, 0)),
        pl.BlockSpec((1, _HIDDEN), lambda i: (0, 0)),
        pl.BlockSpec((_HIDDEN, code), lambda i: (0, 0)),
        pl.BlockSpec((1, code), lambda i: (0, 0)),
    ]
    args = [x, W1, b1[None, :], W2, b2[None, :]]
    kwargs = {}
    if acc is None:
        body = _mlp_body_first
    else:
        body = _mlp_body_acc
        in_specs.append(pl.BlockSpec(memory_space=pltpu.MemorySpace.HBM))
        args.append(acc)
        kwargs["input_output_aliases"] = {5: 0}
    blk0 = go * (_BPG // _BB)
    return pl.pallas_call(
        body,
        grid=(_BPG // _BB,),
        in_specs=in_specs,
        out_specs=pl.BlockSpec((_BB, _L, code), lambda i: (blk0 + i, 0, 0)),
        out_shape=jax.ShapeDtypeStruct((_B, _L, code), jnp.float32),
        **kwargs,
    )(*args)


def kernel(src_seq, emb_table, W1, b1, W2, b2):
    gathered = [g(src_seq, emb_table) for g in _SC_GATHERS]
    out = _mlp(0, gathered[0], W1, b1, W2, b2)
    out = _mlp(1, gathered[1], W1, b1, W2, b2, acc=out)
    return out


# speedup vs baseline: 1.3359x; 1.2672x over previous
import functools

import jax
import jax.numpy as jnp
from jax import lax
from jax.experimental import pallas as pl
from jax.experimental.pallas import tpu as pltpu
from jax.experimental.pallas import tpu_sc as plsc

_HIDDEN = 128
_B = 4096
_L = 50
_N_ROWS = _B * _L

_G = 2
_BPG = _B // _G
_ROWS_G = _BPG * _L
_BSLAB = _BPG // _HIDDEN

_INFO = plsc.get_sparse_core_info()
_NC = _INFO.num_cores
_NS = _INFO.num_subcores
_NW = _NC * _NS
_PER_W = _ROWS_G // _NW
_LPW = _L // 2
_CHUNK = 640
_N_CHUNKS = _PER_W // _CHUNK
_SLABS = _CHUNK // _HIDDEN


def _make_sc_gather(go):
    def body(idx_hbm, table_hbm, out_hbm, idx2_v, idx_v, rows_v, sem):
        wid = lax.axis_index("s") * _NC + lax.axis_index("c")
        wb = lax.rem(wid, _BSLAB)
        base_l = jnp.where(wid >= _BSLAB, _LPW, 0)

        seq_base = go * _BPG + wb * _HIDDEN
        pltpu.sync_copy(idx_hbm.at[pl.ds(seq_base, _HIDDEN), :], idx2_v)

        def flatten(j, carry):
            k = j * 16 + lax.iota(jnp.int32, 16)
            r = k & (_HIDDEN - 1)
            col = base_l + lax.shift_right_logical(k, 7)
            idx_v[pl.ds(j * 16, 16)] = plsc.load_gather(idx2_v, [r, col])
            return carry

        lax.fori_loop(0, _PER_W // 16, flatten, 0, unroll=8)

        def chunk(c, carry):
            off = c * _CHUNK
            pltpu.async_copy(
                table_hbm.at[idx_v.at[pl.ds(off, _CHUNK)]], rows_v, sem
            ).wait()
            for j in range(_SLABS):
                pltpu.sync_copy(
                    rows_v.at[pl.ds(j * _HIDDEN, _HIDDEN), :],
                    out_hbm.at[base_l + c * _SLABS + j, wb],
                )
            return carry

        lax.fori_loop(0, _N_CHUNKS, chunk, 0)

    return functools.partial(
        pl.kernel,
        mesh=plsc.VectorSubcoreMesh(core_axis_name="c", subcore_axis_name="s"),
        out_type=jax.ShapeDtypeStruct((_L, _BSLAB, _HIDDEN, _HIDDEN),
                                      jnp.float32),
        scratch_types=[
            pltpu.VMEM((_HIDDEN, _L), jnp.int32),
            pltpu.VMEM((_PER_W,), jnp.int32),
            pltpu.VMEM((_CHUNK, _HIDDEN), jnp.float32),
            pltpu.SemaphoreType.DMA,
        ],
        compiler_params=pltpu.CompilerParams(
            needs_layout_passes=False, use_tc_tiling_on_sc=True
        ),
    )(body)


_SC_GATHERS = [_make_sc_gather(go) for go in range(_G)]

_BLK = _BPG


def _mlp_compute(x_ref, w1_ref, b1_ref, w2_ref, b2_ref, o_ref):
    x = x_ref[...].reshape(_BLK, _HIDDEN)
    h = jnp.dot(x, w1_ref[...], preferred_element_type=jnp.float32)
    h = jnp.maximum(h + b1_ref[...], 0.0)
    o = jnp.dot(h, w2_ref[...], preferred_element_type=jnp.float32)
    o_ref[...] = jnp.maximum(o + b2_ref[...], 0.0).reshape(o_ref.shape)


def _mlp_body_first(x_ref, w1_ref, b1_ref, w2_ref, b2_ref, o_ref):
    _mlp_compute(x_ref, w1_ref, b1_ref, w2_ref, b2_ref, o_ref)


def _mlp_body_acc(x_ref, w1_ref, b1_ref, w2_ref, b2_ref, acc_ref, o_ref):
    del acc_ref
    _mlp_compute(x_ref, w1_ref, b1_ref, w2_ref, b2_ref, o_ref)


def _mlp(go, x, W1, b1, W2, b2, acc=None):
    code = W2.shape[1]
    in_specs = [
        pl.BlockSpec((1, _BSLAB, _HIDDEN, _HIDDEN), lambda i: (i, 0, 0, 0)),
        pl.BlockSpec((_HIDDEN, _HIDDEN), lambda i: (0, 0)),
        pl.BlockSpec((1, _HIDDEN), lambda i: (0, 0)),
        pl.BlockSpec((_HIDDEN, code), lambda i: (0, 0)),
        pl.BlockSpec((1, code), lambda i: (0, 0)),
    ]
    args = [x, W1, b1[None, :], W2, b2[None, :]]
    kwargs = {}
    if acc is None:
        body = _mlp_body_first
    else:
        body = _mlp_body_acc
        in_specs.append(pl.BlockSpec(memory_space=pltpu.MemorySpace.HBM))
        args.append(acc)
        kwargs["input_output_aliases"] = {5: 0}
    return pl.pallas_call(
        body,
        grid=(_L,),
        in_specs=in_specs,
        out_specs=pl.BlockSpec((1, _BPG, code), lambda i: (i, go, 0)),
        out_shape=jax.ShapeDtypeStruct((_L, _B, code), jnp.float32),
        **kwargs,
    )(*args)


def kernel(src_seq, emb_table, W1, b1, W2, b2):
    gathered = [g(src_seq, emb_table) for g in _SC_GATHERS]
    out_t = _mlp(0, gathered[0], W1, b1, W2, b2)
    out_t = _mlp(1, gathered[1], W1, b1, W2, b2, acc=out_t)
    return out_t.transpose(1, 0, 2)


# speedup vs baseline: 1.5046x; 1.1263x over previous
import functools

import jax
import jax.numpy as jnp
from jax import lax
from jax.experimental import pallas as pl
from jax.experimental.pallas import tpu as pltpu
from jax.experimental.pallas import tpu_sc as plsc

_HIDDEN = 128
_B = 4096
_L = 50
_N_ROWS = _B * _L

_G = 2
_BPG = _B // _G
_ROWS_G = _BPG * _L
_BSLAB = _BPG // _HIDDEN

_INFO = plsc.get_sparse_core_info()
_NC = _INFO.num_cores
_NS = _INFO.num_subcores
_NW = _NC * _NS
_PER_W = _ROWS_G // _NW
_LPW = _L // 2
_CHUNK = 640
_N_CHUNKS = _PER_W // _CHUNK
_SLABS = _CHUNK // _HIDDEN


def _make_sc_gather(go):
    def body(idx_hbm, table_hbm, out_hbm, idx2_v, idx_v, rows_v, sem):
        wid = lax.axis_index("s") * _NC + lax.axis_index("c")
        wb = lax.rem(wid, _BSLAB)
        base_l = jnp.where(wid >= _BSLAB, _LPW, 0)

        seq_base = go * _BPG + wb * _HIDDEN
        pltpu.sync_copy(idx_hbm.at[pl.ds(seq_base, _HIDDEN), :], idx2_v)

        def flatten(j, carry):
            k = j * 16 + lax.iota(jnp.int32, 16)
            r = k & (_HIDDEN - 1)
            col = base_l + lax.shift_right_logical(k, 7)
            idx_v[pl.ds(j * 16, 16)] = plsc.load_gather(idx2_v, [r, col])
            return carry

        lax.fori_loop(0, _PER_W // 16, flatten, 0, unroll=8)

        def chunk(c, carry):
            off = c * _CHUNK
            pltpu.async_copy(
                table_hbm.at[idx_v.at[pl.ds(off, _CHUNK)]], rows_v, sem
            ).wait()
            for j in range(_SLABS):
                pltpu.sync_copy(
                    rows_v.at[pl.ds(j * _HIDDEN, _HIDDEN), :],
                    out_hbm.at[base_l + c * _SLABS + j, wb],
                )
            return carry

        lax.fori_loop(0, _N_CHUNKS, chunk, 0)

    return functools.partial(
        pl.kernel,
        mesh=plsc.VectorSubcoreMesh(core_axis_name="c", subcore_axis_name="s"),
        out_type=jax.ShapeDtypeStruct((_L, _BSLAB, _HIDDEN, _HIDDEN),
                                      jnp.float32),
        scratch_types=[
            pltpu.VMEM((_HIDDEN, _L), jnp.int32),
            pltpu.VMEM((_PER_W,), jnp.int32),
            pltpu.VMEM((_CHUNK, _HIDDEN), jnp.float32),
            pltpu.SemaphoreType.DMA,
        ],
        compiler_params=pltpu.CompilerParams(
            needs_layout_passes=False, use_tc_tiling_on_sc=True
        ),
    )(body)


_SC_GATHERS = [_make_sc_gather(go) for go in range(_G)]

_LB = 2
_BLK = _BPG * _LB


def _mlp_compute(x_ref, w1_ref, b1_ref, w2_ref, b2_ref, o_ref):
    x = x_ref[...].reshape(_BLK, _HIDDEN)
    h = jnp.dot(x, w1_ref[...], preferred_element_type=jnp.float32)
    h = jnp.maximum(h + b1_ref[...], 0.0)
    o = jnp.dot(h, w2_ref[...], preferred_element_type=jnp.float32)
    o_ref[...] = jnp.maximum(o + b2_ref[...], 0.0).reshape(o_ref.shape)


def _mlp_body_first(x_ref, w1_ref, b1_ref, w2_ref, b2_ref, o_ref):
    _mlp_compute(x_ref, w1_ref, b1_ref, w2_ref, b2_ref, o_ref)


def _mlp_body_acc(x_ref, w1_ref, b1_ref, w2_ref, b2_ref, acc_ref, o_ref):
    del acc_ref
    _mlp_compute(x_ref, w1_ref, b1_ref, w2_ref, b2_ref, o_ref)


def _mlp(go, x, W1, b1, W2, b2, acc=None):
    code = W2.shape[1]
    in_specs = [
        pl.BlockSpec((_LB, _BSLAB, _HIDDEN, _HIDDEN), lambda i: (i, 0, 0, 0)),
        pl.BlockSpec((_HIDDEN, _HIDDEN), lambda i: (0, 0)),
        pl.BlockSpec((1, _HIDDEN), lambda i: (0, 0)),
        pl.BlockSpec((_HIDDEN, code), lambda i: (0, 0)),
        pl.BlockSpec((1, code), lambda i: (0, 0)),
    ]
    args = [x, W1, b1[None, :], W2, b2[None, :]]
    kwargs = {}
    if acc is None:
        body = _mlp_body_first
    else:
        body = _mlp_body_acc
        in_specs.append(pl.BlockSpec(memory_space=pltpu.MemorySpace.HBM))
        args.append(acc)
        kwargs["input_output_aliases"] = {5: 0}
    return pl.pallas_call(
        body,
        grid=(_L // _LB,),
        in_specs=in_specs,
        out_specs=pl.BlockSpec((_LB, _BPG, code), lambda i: (i, go, 0)),
        out_shape=jax.ShapeDtypeStruct((_L, _B, code), jnp.float32),
        **kwargs,
    )(*args)


def kernel(src_seq, emb_table, W1, b1, W2, b2):
    gathered = [g(src_seq, emb_table) for g in _SC_GATHERS]
    out_t = _mlp(0, gathered[0], W1, b1, W2, b2)
    out_t = _mlp(1, gathered[1], W1, b1, W2, b2, acc=out_t)
    return out_t.transpose(1, 0, 2)


# speedup vs baseline: 1.5655x; 1.0405x over previous
import functools

import jax
import jax.numpy as jnp
from jax import lax
from jax.experimental import pallas as pl
from jax.experimental.pallas import tpu as pltpu
from jax.experimental.pallas import tpu_sc as plsc

_HIDDEN = 128
_B = 4096
_L = 50
_N_ROWS = _B * _L

_G = 2
_BPG = _B // _G
_ROWS_G = _BPG * _L
_BSLAB = _BPG // _HIDDEN

_INFO = plsc.get_sparse_core_info()
_NC = _INFO.num_cores
_NS = _INFO.num_subcores
_NW = _NC * _NS
_PER_W = _ROWS_G // _NW
_LPW = _L // 2
_CHUNK = 640
_N_CHUNKS = _PER_W // _CHUNK
_SLABS = _CHUNK // _HIDDEN


def _make_sc_gather(go):
    def body(idx_hbm, table_hbm, out_hbm, idx2_v, idx_v, rows_v, sem):
        wid = lax.axis_index("s") * _NC + lax.axis_index("c")
        wb = lax.rem(wid, _BSLAB)
        base_l = jnp.where(wid >= _BSLAB, _LPW, 0)

        seq_base = go * _BPG + wb * _HIDDEN
        pltpu.sync_copy(idx_hbm.at[pl.ds(seq_base, _HIDDEN), :], idx2_v)

        def flatten(j, carry):
            k = j * 16 + lax.iota(jnp.int32, 16)
            r = k & (_HIDDEN - 1)
            col = base_l + lax.shift_right_logical(k, 7)
            idx_v[pl.ds(j * 16, 16)] = plsc.load_gather(idx2_v, [r, col])
            return carry

        lax.fori_loop(0, _PER_W // 16, flatten, 0, unroll=8)

        def chunk(c, carry):
            off = c * _CHUNK
            pltpu.async_copy(
                table_hbm.at[idx_v.at[pl.ds(off, _CHUNK)]], rows_v, sem
            ).wait()
            for j in range(_SLABS):
                pltpu.sync_copy(
                    rows_v.at[pl.ds(j * _HIDDEN, _HIDDEN), :],
                    out_hbm.at[base_l + c * _SLABS + j, wb],
                )
            return carry

        lax.fori_loop(0, _N_CHUNKS, chunk, 0)

    return functools.partial(
        pl.kernel,
        mesh=plsc.VectorSubcoreMesh(core_axis_name="c", subcore_axis_name="s"),
        out_type=jax.ShapeDtypeStruct((_L, _BSLAB, _HIDDEN, _HIDDEN),
                                      jnp.float32),
        scratch_types=[
            pltpu.VMEM((_HIDDEN, _L), jnp.int32),
            pltpu.VMEM((_PER_W,), jnp.int32),
            pltpu.VMEM((_CHUNK, _HIDDEN), jnp.float32),
            pltpu.SemaphoreType.DMA,
        ],
        compiler_params=pltpu.CompilerParams(
            needs_layout_passes=False, use_tc_tiling_on_sc=True
        ),
    )(body)


_SC_GATHERS = [_make_sc_gather(go) for go in range(_G)]

_LB = 5
_BLK = _BPG * _LB


def _mlp_compute(x_ref, w1_ref, b1_ref, w2_ref, b2_ref, o_ref):
    x = x_ref[...].reshape(_BLK, _HIDDEN)
    h = jnp.dot(x, w1_ref[...], preferred_element_type=jnp.float32)
    h = jnp.maximum(h + b1_ref[...], 0.0)
    o = jnp.dot(h, w2_ref[...], preferred_element_type=jnp.float32)
    o_ref[...] = jnp.maximum(o + b2_ref[...], 0.0).reshape(o_ref.shape)


def _mlp_body_first(x_ref, w1_ref, b1_ref, w2_ref, b2_ref, o_ref):
    _mlp_compute(x_ref, w1_ref, b1_ref, w2_ref, b2_ref, o_ref)


def _mlp_body_acc(x_ref, w1_ref, b1_ref, w2_ref, b2_ref, acc_ref, o_ref):
    del acc_ref
    _mlp_compute(x_ref, w1_ref, b1_ref, w2_ref, b2_ref, o_ref)


def _mlp(go, x, W1, b1, W2, b2, acc=None):
    code = W2.shape[1]
    in_specs = [
        pl.BlockSpec((_LB, _BSLAB, _HIDDEN, _HIDDEN), lambda i: (i, 0, 0, 0)),
        pl.BlockSpec((_HIDDEN, _HIDDEN), lambda i: (0, 0)),
        pl.BlockSpec((1, _HIDDEN), lambda i: (0, 0)),
        pl.BlockSpec((_HIDDEN, code), lambda i: (0, 0)),
        pl.BlockSpec((1, code), lambda i: (0, 0)),
    ]
    args = [x, W1, b1[None, :], W2, b2[None, :]]
    kwargs = {}
    if acc is None:
        body = _mlp_body_first
    else:
        body = _mlp_body_acc
        in_specs.append(pl.BlockSpec(memory_space=pltpu.MemorySpace.HBM))
        args.append(acc)
        kwargs["input_output_aliases"] = {5: 0}
    return pl.pallas_call(
        body,
        grid=(_L // _LB,),
        in_specs=in_specs,
        out_specs=pl.BlockSpec((_LB, _BPG, code), lambda i: (i, go, 0)),
        out_shape=jax.ShapeDtypeStruct((_L, _B, code), jnp.float32),
        **kwargs,
    )(*args)


def kernel(src_seq, emb_table, W1, b1, W2, b2):
    gathered = [g(src_seq, emb_table) for g in _SC_GATHERS]
    out_t = _mlp(0, gathered[0], W1, b1, W2, b2)
    out_t = _mlp(1, gathered[1], W1, b1, W2, b2, acc=out_t)
    return out_t.transpose(1, 0, 2)


# speedup vs baseline: 1.5996x; 1.0218x over previous
import functools

import jax
import jax.numpy as jnp
from jax import lax
from jax.experimental import pallas as pl
from jax.experimental.pallas import tpu as pltpu
from jax.experimental.pallas import tpu_sc as plsc

_HIDDEN = 128
_B = 4096
_L = 50
_N_ROWS = _B * _L

_G = 2
_BPG = _B // _G
_ROWS_G = _BPG * _L
_BSLAB = _BPG // _HIDDEN

_INFO = plsc.get_sparse_core_info()
_NC = _INFO.num_cores
_NS = _INFO.num_subcores
_NW = _NC * _NS
_PER_W = _ROWS_G // _NW
_LPW = _L // 2
_CHUNK = 640
_N_CHUNKS = _PER_W // _CHUNK
_SLABS = _CHUNK // _HIDDEN


def _make_sc_gather(go):
    def body(idx_hbm, table_hbm, out_hbm, idx2_v, idx_v, rows_v, sem):
        wid = lax.axis_index("s") * _NC + lax.axis_index("c")
        wb = lax.rem(wid, _BSLAB)
        base_l = jnp.where(wid >= _BSLAB, _LPW, 0)

        seq_base = go * _BPG + wb * _HIDDEN
        pltpu.sync_copy(idx_hbm.at[pl.ds(seq_base, _HIDDEN), :], idx2_v)

        def flatten(j, carry):
            k = j * 16 + lax.iota(jnp.int32, 16)
            r = k & (_HIDDEN - 1)
            col = base_l + lax.shift_right_logical(k, 7)
            idx_v[pl.ds(j * 16, 16)] = plsc.load_gather(idx2_v, [r, col])
            return carry

        lax.fori_loop(0, _PER_W // 16, flatten, 0, unroll=8)

        def chunk(c, carry):
            off = c * _CHUNK
            pltpu.async_copy(
                table_hbm.at[idx_v.at[pl.ds(off, _CHUNK)]], rows_v, sem
            ).wait()
            for j in range(_SLABS):
                pltpu.sync_copy(
                    rows_v.at[pl.ds(j * _HIDDEN, _HIDDEN), :],
                    out_hbm.at[base_l + c * _SLABS + j, wb],
                )
            return carry

        lax.fori_loop(0, _N_CHUNKS, chunk, 0)

    return functools.partial(
        pl.kernel,
        mesh=plsc.VectorSubcoreMesh(core_axis_name="c", subcore_axis_name="s"),
        out_type=jax.ShapeDtypeStruct((_L, _BSLAB, _HIDDEN, _HIDDEN),
                                      jnp.float32),
        scratch_types=[
            pltpu.VMEM((_HIDDEN, _L), jnp.int32),
            pltpu.VMEM((_PER_W,), jnp.int32),
            pltpu.VMEM((_CHUNK, _HIDDEN), jnp.float32),
            pltpu.SemaphoreType.DMA,
        ],
        compiler_params=pltpu.CompilerParams(
            needs_layout_passes=False, use_tc_tiling_on_sc=True
        ),
    )(body)


_SC_GATHERS = [_make_sc_gather(go) for go in range(_G)]

_LB = 10
_BLK = _BPG * _LB


def _mlp_compute(x_ref, w1_ref, b1_ref, w2_ref, b2_ref, o_ref):
    x = x_ref[...].reshape(_BLK, _HIDDEN)
    h = jnp.dot(x, w1_ref[...], preferred_element_type=jnp.float32)
    h = jnp.maximum(h + b1_ref[...], 0.0)
    o = jnp.dot(h, w2_ref[...], preferred_element_type=jnp.float32)
    o_ref[...] = jnp.maximum(o + b2_ref[...], 0.0).reshape(o_ref.shape)


def _mlp_body_first(x_ref, w1_ref, b1_ref, w2_ref, b2_ref, o_ref):
    _mlp_compute(x_ref, w1_ref, b1_ref, w2_ref, b2_ref, o_ref)


def _mlp_body_acc(x_ref, w1_ref, b1_ref, w2_ref, b2_ref, acc_ref, o_ref):
    del acc_ref
    _mlp_compute(x_ref, w1_ref, b1_ref, w2_ref, b2_ref, o_ref)


def _mlp(go, x, W1, b1, W2, b2, acc=None):
    code = W2.shape[1]
    in_specs = [
        pl.BlockSpec((_LB, _BSLAB, _HIDDEN, _HIDDEN), lambda i: (i, 0, 0, 0)),
        pl.BlockSpec((_HIDDEN, _HIDDEN), lambda i: (0, 0)),
        pl.BlockSpec((1, _HIDDEN), lambda i: (0, 0)),
        pl.BlockSpec((_HIDDEN, code), lambda i: (0, 0)),
        pl.BlockSpec((1, code), lambda i: (0, 0)),
    ]
    args = [x, W1, b1[None, :], W2, b2[None, :]]
    kwargs = {}
    if acc is None:
        body = _mlp_body_first
    else:
        body = _mlp_body_acc
        in_specs.append(pl.BlockSpec(memory_space=pltpu.MemorySpace.HBM))
        args.append(acc)
        kwargs["input_output_aliases"] = {5: 0}
    return pl.pallas_call(
        body,
        grid=(_L // _LB,),
        in_specs=in_specs,
        out_specs=pl.BlockSpec((_LB, _BPG, code), lambda i: (i, go, 0)),
        out_shape=jax.ShapeDtypeStruct((_L, _B, code), jnp.float32),
        **kwargs,
    )(*args)


def kernel(src_seq, emb_table, W1, b1, W2, b2):
    gathered = [g(src_seq, emb_table) for g in _SC_GATHERS]
    out_t = _mlp(0, gathered[0], W1, b1, W2, b2)
    out_t = _mlp(1, gathered[1], W1, b1, W2, b2, acc=out_t)
    return out_t.transpose(1, 0, 2)
